# Initial kernel scaffold; baseline (speedup 1.0000x reference)
import jax
import jax.numpy as jnp
from jax.experimental import pallas as pl


def kernel(x, edge_index, W1, as1, ad1, b1, W2, as2, ad2, b2):
    raise NotImplementedError("write your pallas kernel here")


# speedup vs baseline: 59.4945x; 59.4945x over previous
import functools

import jax
import jax.numpy as jnp
from jax import lax
from jax.experimental import pallas as pl
from jax.experimental.pallas import tpu as pltpu
from jax.experimental.pallas import tpu_sc as plsc

N_NODES = 10000
NPAD = 10240
F_IN = 128
C = 128
TILES = 32
EPT = 10368
G = EPT // C
E_PAD = TILES * EPT
RB = 512


def _edge_pass(srcv, dstv, att, feat, heads, cols, tblw):
    ch_iters = C * heads // 16
    kv = cols // 16
    rps = NPAD // 16
    zr = 128
    span = cols // heads
    mesh = plsc.VectorSubcoreMesh(
        core_axis_name="c", subcore_axis_name="s", num_cores=2, num_subcores=16)

    @functools.partial(
        pl.kernel,
        out_type=jax.ShapeDtypeStruct((2, NPAD, tblw), jnp.float32),
        mesh=mesh,
        compiler_params=pltpu.CompilerParams(
            needs_layout_passes=False, use_tc_tiling_on_sc=False),
        scratch_types=[
            pltpu.VMEM((C,), jnp.int32),
            pltpu.VMEM((C,), jnp.int32),
            pltpu.VMEM((C, 16), jnp.float32),
            pltpu.VMEM((C, 16), jnp.float32),
            pltpu.VMEM((C, cols), jnp.float32),
            pltpu.VMEM((C, tblw), jnp.float32),
            pltpu.VMEM((128, tblw), jnp.float32),
            pltpu.VMEM_SHARED((NPAD, tblw), jnp.float32),
            pltpu.SemaphoreType.DMA,
            pltpu.SemaphoreType.DMA,
        ],
    )
    def k(src_hbm, dst_hbm, a_hbm, h_hbm, out_hbm,
          v_is, v_id, v_gs, v_gd, v_h, v_cb, v_z, s_tbl, sem_a, sem_b):
        cid = lax.axis_index("c")
        sid = lax.axis_index("s")
        tile = cid * 16 + sid
        z16 = jnp.zeros((16,), jnp.float32)
        iot = lax.iota(jnp.int32, 16)
        nk = tblw // 16

        def zero_buf(ref, rows):
            def zb(i, _):
                ref[i // nk, pl.ds((i % nk) * 16, 16)] = z16
                return 0
            lax.fori_loop(0, rows * nk, zb, 0)

        zero_buf(v_z, 128)
        zero_buf(v_cb, C)
        for j in range(rps // 128):
            pltpu.sync_copy(v_z, s_tbl.at[pl.ds(sid * rps + j * 128, 128)])
        plsc.subcore_barrier()

        def chunk(g, _):
            ebase = tile * EPT + g * C
            cp1 = pltpu.async_copy(src_hbm.at[pl.ds(ebase, C)], v_is, sem_a)
            cp2 = pltpu.async_copy(dst_hbm.at[pl.ds(ebase, C)], v_id, sem_a)
            cp1.wait()
            cp2.wait()
            g1 = pltpu.async_copy(a_hbm.at[v_is], v_gs, sem_b)
            g2 = pltpu.async_copy(a_hbm.at[v_id], v_gd, sem_b)
            g3 = pltpu.async_copy(h_hbm.at[v_is], v_h, sem_b)
            g1.wait()
            g2.wait()
            g3.wait()

            def wp(t, _):
                pv = t * 16 + iot
                if heads == 8:
                    row = pv >> 3
                    hd = pv & 7
                else:
                    row = pv
                    hd = iot & 0
                av = plsc.load_gather(v_gs, [row, hd])
                bv = plsc.load_gather(v_gd, [row, hd + 8])
                e = av + bv
                e = jnp.where(e > 0, e, 0.2 * e)
                plsc.store_scatter(v_cb, [row, hd], jnp.exp(e))
                return 0
            lax.fori_loop(0, ch_iters, wp, 0)

            def mp(c, _):
                cs = jnp.zeros((16,), jnp.int32) + c
                for k2 in range(kv):
                    tmpl = (k2 * 16 + iot) >> (3 if span == 8 else 4)
                    wv = plsc.load_gather(v_cb, [cs, tmpl])
                    hv = v_h[c, pl.ds(k2 * 16, 16)]
                    v_cb[c, pl.ds(8 + k2 * 16, 16)] = wv * hv
                return 0
            lax.fori_loop(0, C, mp, 0)

            pltpu.sync_copy(v_cb, s_tbl.at[v_id], add=True)
            return 0
        lax.fori_loop(0, G, chunk, 0)
        plsc.subcore_barrier()
        for j in range(rps // 128):
            r0 = pl.multiple_of(sid * rps + j * 128, 128)
            pltpu.sync_copy(s_tbl.at[pl.ds(r0, 128)], v_z)
            pltpu.sync_copy(v_z, out_hbm.at[cid, pl.ds(r0, 128)])

    return k(srcv, dstv, att, feat)


def _k1_body(x_ref, w_ref, s_ref, h_ref, a_ref):
    h = jnp.dot(x_ref[...], w_ref[...], preferred_element_type=jnp.float32)
    h_ref[...] = h
    a_ref[...] = jnp.dot(h, s_ref[...], preferred_element_type=jnp.float32)


def _k2_body(p_ref, er_ref, b_ref, w2_ref, s2_ref, h2_ref, a2_ref):
    t = p_ref[0] + p_ref[1]
    den = jnp.dot(t[:, 0:8], er_ref[...], preferred_element_type=jnp.float32)
    out1 = t[:, 8:72] / (den + 1e-16) + b_ref[...]
    out1 = jnp.where(out1 > 0, out1, 0.2 * out1)
    h2 = jnp.dot(out1, w2_ref[...], preferred_element_type=jnp.float32)
    h2_ref[...] = h2
    a2_ref[...] = jnp.dot(h2, s2_ref[...], preferred_element_type=jnp.float32)


def _k3_body(p_ref, b_ref, o_ref):
    t = p_ref[0] + p_ref[1]
    z = t[:, 8:24] / (t[:, 0:1] + 1e-16) + b_ref[...]
    m = jnp.max(z, axis=1, keepdims=True)
    o_ref[...] = z - m - jnp.log(jnp.sum(jnp.exp(z - m), axis=1, keepdims=True))


def kernel(x, edge_index, W1, as1, ad1, b1, W2, as2, ad2, b2):
    n = x.shape[0]
    loops = jnp.arange(n, dtype=jnp.int32)
    pad = jnp.full((E_PAD - edge_index.shape[1] - n,), N_NODES, jnp.int32)
    src = jnp.concatenate([edge_index[0].astype(jnp.int32), loops, pad])
    dst = jnp.concatenate([edge_index[1].astype(jnp.int32), loops, pad])
    x_pad = jnp.zeros((NPAD, F_IN), jnp.float32).at[:n].set(x)

    s1 = jnp.zeros((64, 16), jnp.float32)
    for hd in range(8):
        s1 = s1.at[hd * 8:(hd + 1) * 8, hd].set(as1[hd])
        s1 = s1.at[hd * 8:(hd + 1) * 8, 8 + hd].set(ad1[hd])
    s2 = jnp.zeros((16, 16), jnp.float32).at[:, 0].set(as2[0]).at[:, 8].set(ad2[0])
    erep = jnp.repeat(jnp.eye(8, dtype=jnp.float32), 8, axis=1)

    grid = (NPAD // RB,)
    h1, a1 = pl.pallas_call(
        _k1_body,
        grid=grid,
        in_specs=[
            pl.BlockSpec((RB, F_IN), lambda i: (i, 0)),
            pl.BlockSpec((F_IN, 64), lambda i: (0, 0)),
            pl.BlockSpec((64, 16), lambda i: (0, 0)),
        ],
        out_specs=[
            pl.BlockSpec((RB, 64), lambda i: (i, 0)),
            pl.BlockSpec((RB, 16), lambda i: (i, 0)),
        ],
        out_shape=[
            jax.ShapeDtypeStruct((NPAD, 64), jnp.float32),
            jax.ShapeDtypeStruct((NPAD, 16), jnp.float32),
        ],
    )(x_pad, W1, s1)

    p1 = _edge_pass(src, dst, a1, h1, heads=8, cols=64, tblw=80)

    h2, a2 = pl.pallas_call(
        _k2_body,
        grid=grid,
        in_specs=[
            pl.BlockSpec((2, RB, 80), lambda i: (0, i, 0)),
            pl.BlockSpec((8, 64), lambda i: (0, 0)),
            pl.BlockSpec((1, 64), lambda i: (0, 0)),
            pl.BlockSpec((64, 16), lambda i: (0, 0)),
            pl.BlockSpec((16, 16), lambda i: (0, 0)),
        ],
        out_specs=[
            pl.BlockSpec((RB, 16), lambda i: (i, 0)),
            pl.BlockSpec((RB, 16), lambda i: (i, 0)),
        ],
        out_shape=[
            jax.ShapeDtypeStruct((NPAD, 16), jnp.float32),
            jax.ShapeDtypeStruct((NPAD, 16), jnp.float32),
        ],
    )(p1, erep, b1.reshape(1, 64), W2, s2)

    p2 = _edge_pass(src, dst, a2, h2, heads=1, cols=16, tblw=32)

    out = pl.pallas_call(
        _k3_body,
        grid=grid,
        in_specs=[
            pl.BlockSpec((2, RB, 32), lambda i: (0, i, 0)),
            pl.BlockSpec((1, 16), lambda i: (0, 0)),
        ],
        out_specs=pl.BlockSpec((RB, 16), lambda i: (i, 0)),
        out_shape=jax.ShapeDtypeStruct((NPAD, 16), jnp.float32),
    )(p2, b2.reshape(1, 16))
    return out[:n]
